# baseline (speedup 1.0000x reference)
import functools

import jax
import jax.numpy as jnp
from jax import lax
from jax.experimental import pallas as pl
from jax.experimental.pallas import tpu as pltpu
from jax.experimental.pallas import tpu_sc as plsc

N = 10000
E = 320000
D = 128
H1, C1 = 8, 16
C2 = 16

NC, NS = 2, 16
NW = NC * NS
B1 = 80
B2 = 128
STRIPE = 624
TAIL = N - NS * STRIPE
TAIL_OFF = NS * STRIPE


def _contract(W, att, heads, ch):
    return jnp.einsum('dhc,hc->dh', W.reshape(W.shape[0], heads, ch), att[0])


def _prep_body(x_ref, w_ref, t_ref, ad_ref):
    out = jnp.dot(x_ref[...], w_ref[...], preferred_element_type=jnp.float32)
    t_ref[...] = out[:, :144]
    ad_ref[...] = out[:, 144:]


def _prep_tables(x, wcat):
    nblk = 1000
    return pl.pallas_call(
        _prep_body,
        grid=(N // nblk,),
        in_specs=[
            pl.BlockSpec((nblk, 128), lambda i: (i, 0)),
            pl.BlockSpec((128, 160), lambda i: (0, 0)),
        ],
        out_specs=[
            pl.BlockSpec((nblk, 144), lambda i: (i, 0)),
            pl.BlockSpec((nblk, 16), lambda i: (i, 0)),
        ],
        out_shape=[
            jax.ShapeDtypeStruct((N, 144), jnp.float32),
            jax.ShapeDtypeStruct((N, 16), jnp.float32),
        ],
    )(x, wcat)


def _ae_body(ea_ref, ma_ref, o_ref):
    a = pl.program_id(1)
    q = jnp.dot(ea_ref[0], ma_ref[0], preferred_element_type=jnp.float32)

    @pl.when(a == 0)
    def _():
        o_ref[...] = q

    @pl.when(a > 0)
    def _():
        o_ref[...] += q


def _ae_pack(eat3, ma):
    eblk = 1600
    return pl.pallas_call(
        _ae_body,
        grid=(E // 8 // eblk, 16),
        in_specs=[
            pl.BlockSpec((1, eblk, 8), lambda i, a: (a, i, 0)),
            pl.BlockSpec((1, 8, 128), lambda i, a: (a, 0, 0)),
        ],
        out_specs=pl.BlockSpec((eblk, 128), lambda i, a: (i, 0)),
        out_shape=jax.ShapeDtypeStruct((E // 8, 128), jnp.float32),
    )(eat3, ma)


def _layer1_finish_body(accm_ref, accd_ref, b1_ref, w_ref, t2_ref, ad2_ref):
    num = (accm_ref[0] + accm_ref[1]).reshape(-1, H1, C1)
    dsum = accd_ref[0] + accd_ref[1]
    den = dsum[:, :8].reshape(-1, H1, 1)
    v = (num / (den + 1e-16)).reshape(-1, 128) + b1_ref[0]
    h1 = jnp.where(v > 0, v, jnp.exp(v) - 1.0)
    out = jnp.dot(h1, w_ref[...], preferred_element_type=jnp.float32)
    t2_ref[...] = out[:, :32]
    ad2_ref[...] = out[:, 32:]


def _layer1_finish(accm1, accd1, b1, wcat2):
    blk = 1000
    return pl.pallas_call(
        _layer1_finish_body,
        grid=(N // blk,),
        in_specs=[
            pl.BlockSpec((2, blk, 128), lambda i: (0, i, 0)),
            pl.BlockSpec((2, blk, 16), lambda i: (0, i, 0)),
            pl.BlockSpec((1, 128), lambda i: (0, 0)),
            pl.BlockSpec((128, 48), lambda i: (0, 0)),
        ],
        out_specs=[
            pl.BlockSpec((blk, 32), lambda i: (i, 0)),
            pl.BlockSpec((blk, 16), lambda i: (i, 0)),
        ],
        out_shape=[
            jax.ShapeDtypeStruct((N, 32), jnp.float32),
            jax.ShapeDtypeStruct((N, 16), jnp.float32),
        ],
    )(accm1, accd1, b1.reshape(1, 128), wcat2)


def _final_body(accp_ref, b2_ref, o_ref):
    acc = accp_ref[0] + accp_ref[1]
    z = acc[:, :16] / (acc[:, 24:25] + 1e-16) + b2_ref[0]
    m = jnp.max(z, axis=1, keepdims=True)
    zz = z - m
    lse = jnp.log(jnp.sum(jnp.exp(zz), axis=1, keepdims=True))
    o_ref[...] = zz - lse


def _final(accp2, b2):
    blk = 1000
    return pl.pallas_call(
        _final_body,
        grid=(N // blk,),
        in_specs=[
            pl.BlockSpec((2, blk, 32), lambda i: (0, i, 0)),
            pl.BlockSpec((1, 16), lambda i: (0, 0)),
        ],
        out_specs=pl.BlockSpec((blk, 16), lambda i: (i, 0)),
        out_shape=jax.ShapeDtypeStruct((N, 16), jnp.float32),
    )(accp2, b2.reshape(1, 16))


def _edge_phase(table, adst, eidx3, ae_pk, width, logit_off, head_pairs,
                Bp, cps, nch, rem_tiles, split_out=False):
    mesh = plsc.VectorSubcoreMesh(core_axis_name="c", subcore_axis_name="s")

    if split_out:
        out_type = [jax.ShapeDtypeStruct((NC, N, logit_off), jnp.float32),
                    jax.ShapeDtypeStruct((NC, N, width - logit_off), jnp.float32)]
    else:
        out_type = jax.ShapeDtypeStruct((NC, N, width), jnp.float32)

    @functools.partial(
        pl.kernel,
        out_type=out_type,
        mesh=mesh,
        compiler_params=pltpu.CompilerParams(use_tc_tiling_on_sc=False),
    scratch_types=[
            pltpu.VMEM((2, cps, Bp), jnp.int32),
            pltpu.VMEM((2, cps, Bp), jnp.int32),
            pltpu.VMEM((cps * Bp // 8, 128), jnp.float32),
            pltpu.VMEM((Bp, width), jnp.float32),
            pltpu.VMEM((Bp, width), jnp.float32),
            pltpu.VMEM((Bp, 16), jnp.float32),
            pltpu.VMEM((Bp, 16), jnp.float32),
            pltpu.VMEM_SHARED((N, width), jnp.float32),
            pltpu.SemaphoreType.DMA,
            pltpu.SemaphoreType.DMA,
            pltpu.SemaphoreType.DMA,
            pltpu.SemaphoreType.DMA,
        ],
    )
    def k(tab, ad, eidx, aer, *rest):
        nout = 2 if split_out else 1
        outs = rest[:nout]
        (sciA, sciB, scae, rows0, rows1, d0, d1, shacc,
         semg0, semg1, sems0, sems1) = rest[nout:]
        c = lax.axis_index("c")
        s = lax.axis_index("s")

        zv = jnp.zeros((16,), jnp.float32)

        @plsc.parallel_loop(0, Bp, unroll=4)
        def _(e):
            for g in range(width // 16):
                rows0[e, pl.ds(g * 16, 16)] = zv

        nz = STRIPE // Bp
        rz = STRIPE - nz * Bp

        @pl.loop(0, nz)
        def _(i):
            pltpu.sync_copy(rows0, shacc.at[pl.ds(s * STRIPE + i * Bp, Bp)])

        if rz:
            pltpu.sync_copy(rows0.at[pl.ds(0, rz)],
                            shacc.at[pl.ds(s * STRIPE + nz * Bp, rz)])

        @pl.when(s == 0)
        def _():
            pltpu.sync_copy(rows0.at[pl.ds(0, TAIL)],
                            shacc.at[pl.ds(TAIL_OFF, TAIL)])

        plsc.subcore_barrier()
        tile = c * NS + s
        chunk0 = tile * nch + jnp.minimum(tile, rem_tiles)

        rpc = Bp // 8
        U = 2 * cps
        M = ((nch - cps) // U) * U
        TL = nch - M
        assert 0 < TL <= cps and (M // cps) % 2 == 0

        def copy_superchunk(sc, sci):
            g = chunk0 + sc * cps
            pltpu.sync_copy(eidx.at[:, pl.ds(g, cps)], sci)
            pltpu.sync_copy(aer.at[pl.ds(g * rpc, cps * rpc)], scae)

        def start_gather(m, sci, bufs):
            rows, d, gsem, _ = bufs
            pltpu.async_copy(tab.at[sci.at[0, m]], rows, gsem)
            pltpu.async_copy(ad.at[sci.at[1, m]], d, gsem)

        def wait_gather(bufs):
            rows, d, gsem, _ = bufs
            pltpu.make_async_copy(tab.at[sciA.at[0, 0]], rows, gsem).wait()
            pltpu.make_async_copy(ad.at[sciA.at[1, 0]], d, gsem).wait()

        def wait_scatter(bufs):
            rows, _, _, ssem = bufs
            pltpu.make_async_copy(rows, shacc.at[sciA.at[1, 0]], ssem).wait()

        def compute_scatter(m, sci, bufs, sync=False):
            rows, d, _, ssem = bufs

            @plsc.parallel_loop(0, Bp, unroll=4)
            def _(e):
                aev = scae[m * rpc + lax.div(e, 8), pl.ds(lax.rem(e, 8) * 16, 16)]
                logit = rows[e, pl.ds(logit_off, 16)] + d[e, :] + aev
                l = jnp.where(logit > 0, logit, logit * 0.2)
                ex = jnp.exp(l)
                rows[e, pl.ds(logit_off, 16)] = ex
                for grp, hl in head_pairs:
                    rows[e, pl.ds(grp * 16, 16)] = rows[e, pl.ds(grp * 16, 16)] * ex[hl]

            if sync:
                pltpu.sync_copy(rows, shacc.at[sci.at[1, m]], add=True)
            else:
                pltpu.async_copy(rows, shacc.at[sci.at[1, m]], ssem, add=True)

        bufA = (rows0, d0, semg0, sems0)
        bufB = (rows1, d1, semg1, sems1)
        copy_superchunk(0, sciA)
        start_gather(0, sciA, bufA)

        @pl.loop(0, M, step=U)
        def _(kk):
            for p in range(U):
                sci = sciA if p < cps else sciB
                bufs, obufs = (bufA, bufB) if p % 2 == 0 else (bufB, bufA)
                m = p % cps
                wait_gather(bufs)
                compute_scatter(m, sci, bufs)
                if p == 0:
                    @pl.when(kk > 0)
                    def _():
                        wait_scatter(obufs)
                else:
                    wait_scatter(obufs)
                if p == cps - 1:
                    copy_superchunk(lax.div(kk, cps) + 1, sciB)
                    start_gather(0, sciB, obufs)
                elif p == U - 1:
                    copy_superchunk(lax.div(kk, cps) + 2, sciA)
                    start_gather(0, sciA, obufs)
                else:
                    start_gather(m + 1, sci, obufs)

        for t in range(TL):
            bufs, obufs = (bufA, bufB) if t % 2 == 0 else (bufB, bufA)
            wait_gather(bufs)
            compute_scatter(t, sciA, bufs)
            if t < TL - 1:
                wait_scatter(obufs)
                start_gather(t + 1, sciA, obufs)

        wait_scatter(bufA)
        wait_scatter(bufB)

        if rem_tiles:
            @pl.when(tile < rem_tiles)
            def _():
                g = chunk0 + nch
                pltpu.sync_copy(eidx.at[:, pl.ds(g, 1)], sciA.at[:, pl.ds(0, 1)])
                pltpu.sync_copy(aer.at[pl.ds(g * rpc, rpc)], scae.at[pl.ds(0, rpc)])
                pltpu.sync_copy(tab.at[sciA.at[0, 0]], rows0)
                pltpu.sync_copy(ad.at[sciA.at[1, 0]], d0)
                compute_scatter(0, sciA, bufA, sync=True)

        plsc.subcore_barrier()

        def copy_out(lo, n):
            if split_out:
                pltpu.sync_copy(shacc.at[pl.ds(lo, n), pl.ds(0, logit_off)],
                                outs[0].at[c, pl.ds(lo, n)])
                pltpu.sync_copy(shacc.at[pl.ds(lo, n), pl.ds(logit_off, width - logit_off)],
                                outs[1].at[c, pl.ds(lo, n)])
            else:
                pltpu.sync_copy(shacc.at[pl.ds(lo, n)],
                                outs[0].at[c, pl.ds(lo, n)])

        copy_out(s * STRIPE, STRIPE)

        @pl.when(s == 0)
        def _():
            copy_out(TAIL_OFF, TAIL)

    return k(table, adst, eidx3, ae_pk)


def kernel(x, edge_index, edge_attr, W1, att_src1, att_dst1, We1, att_edge1,
           b1, W2, att_src2, att_dst2, We2, att_edge2, b2):
    f32 = jnp.float32
    Wsrc1 = _contract(W1, att_src1, H1, C1)
    Wdst1 = _contract(W1, att_dst1, H1, C1)
    Me1 = _contract(We1, att_edge1, H1, C1)
    Wsrc2 = _contract(W2, att_src2, 1, C2)
    Wdst2 = _contract(W2, att_dst2, 1, C2)
    Me2 = _contract(We2, att_edge2, 1, C2)
    z8 = jnp.zeros((D, 8), f32)
    z7 = jnp.zeros((D, 7), f32)
    wcat1 = jnp.concatenate([W1, Wsrc1, z8, Wdst1, z8], axis=1)
    m16 = jnp.concatenate([Me1, Me2, jnp.zeros((16, 7), f32)], axis=1)
    ma = (jnp.eye(8, dtype=f32)[None, :, :, None]
          * m16[:, None, None, :]).reshape(16, 8, 128)
    wcat2 = jnp.concatenate([W2, z8, Wsrc2, z7, z8, Wdst2, z7], axis=1)

    ei32 = edge_index.astype(jnp.int32)
    eat3 = edge_attr.T.reshape(16, E // 8, 8)

    table1, adst1 = _prep_tables(x, wcat1)
    ae_pk = _ae_pack(eat3, ma)

    accm1, accd1 = _edge_phase(table1, adst1,
                               ei32.reshape(2, E // B1, B1), ae_pk,
                               width=144, logit_off=128,
                               head_pairs=[(h, h) for h in range(8)],
                               Bp=B1, cps=5, nch=(E // B1) // NW, rem_tiles=0,
                               split_out=True)

    table2, adst2 = _layer1_finish(accm1, accd1, b1, wcat2)

    nch2 = (E // B2) // NW
    accp2 = _edge_phase(table2, adst2,
                        ei32.reshape(2, E // B2, B2), ae_pk,
                        width=32, logit_off=16, head_pairs=[(0, 8)],
                        Bp=B2, cps=6, nch=nch2,
                        rem_tiles=(E // B2) - nch2 * NW)

    return _final(accp2, b2)


# speedup vs baseline: 1.6971x; 1.6971x over previous
import functools

import jax
import jax.numpy as jnp
from jax import lax
from jax.experimental import pallas as pl
from jax.experimental.pallas import tpu as pltpu
from jax.experimental.pallas import tpu_sc as plsc

N = 10000
E = 320000
D = 128
H1, C1 = 8, 16
C2 = 16

NC, NS = 2, 16
NW = NC * NS
B1 = 80
B2 = 128
STRIPE = 624
TAIL = N - NS * STRIPE
TAIL_OFF = NS * STRIPE


def _contract(W, att, heads, ch):
    return jnp.einsum('dhc,hc->dh', W.reshape(W.shape[0], heads, ch), att[0])


def _prep_body(x_ref, w_ref, ea8_ref, m_ref, t_ref, ad_ref, ae_ref):
    out = jnp.dot(x_ref[...], w_ref[...], preferred_element_type=jnp.float32)
    t_ref[...] = out[:, :144]
    ad_ref[...] = out[:, 144:]
    ae_ref[...] = jnp.dot(ea8_ref[...], m_ref[...],
                          preferred_element_type=jnp.float32)


def _prep_tables(x, wcat, ea8, mblk):
    nblk, eblk = 400, 1600
    return pl.pallas_call(
        _prep_body,
        grid=(N // nblk,),
        in_specs=[
            pl.BlockSpec((nblk, 128), lambda i: (i, 0)),
            pl.BlockSpec((128, 160), lambda i: (0, 0)),
            pl.BlockSpec((eblk, 128), lambda i: (i, 0)),
            pl.BlockSpec((128, 128), lambda i: (0, 0)),
        ],
        out_specs=[
            pl.BlockSpec((nblk, 144), lambda i: (i, 0)),
            pl.BlockSpec((nblk, 16), lambda i: (i, 0)),
            pl.BlockSpec((eblk, 128), lambda i: (i, 0)),
        ],
        out_shape=[
            jax.ShapeDtypeStruct((N, 144), jnp.float32),
            jax.ShapeDtypeStruct((N, 16), jnp.float32),
            jax.ShapeDtypeStruct((E // 8, 128), jnp.float32),
        ],
    )(x, wcat, ea8, mblk)


def _layer1_finish_body(accm_ref, accd_ref, b1_ref, w_ref, t2_ref, ad2_ref):
    num = (accm_ref[0] + accm_ref[1]).reshape(-1, H1, C1)
    dsum = accd_ref[0] + accd_ref[1]
    den = dsum[:, :8].reshape(-1, H1, 1)
    v = (num / (den + 1e-16)).reshape(-1, 128) + b1_ref[0]
    h1 = jnp.where(v > 0, v, jnp.exp(v) - 1.0)
    out = jnp.dot(h1, w_ref[...], preferred_element_type=jnp.float32)
    t2_ref[...] = out[:, :32]
    ad2_ref[...] = out[:, 32:]


def _layer1_finish(accm1, accd1, b1, wcat2):
    blk = 1000
    return pl.pallas_call(
        _layer1_finish_body,
        grid=(N // blk,),
        in_specs=[
            pl.BlockSpec((2, blk, 128), lambda i: (0, i, 0)),
            pl.BlockSpec((2, blk, 16), lambda i: (0, i, 0)),
            pl.BlockSpec((1, 128), lambda i: (0, 0)),
            pl.BlockSpec((128, 48), lambda i: (0, 0)),
        ],
        out_specs=[
            pl.BlockSpec((blk, 32), lambda i: (i, 0)),
            pl.BlockSpec((blk, 16), lambda i: (i, 0)),
        ],
        out_shape=[
            jax.ShapeDtypeStruct((N, 32), jnp.float32),
            jax.ShapeDtypeStruct((N, 16), jnp.float32),
        ],
    )(accm1, accd1, b1.reshape(1, 128), wcat2)


def _final_body(accp_ref, b2_ref, o_ref):
    acc = accp_ref[0] + accp_ref[1]
    z = acc[:, :16] / (acc[:, 24:25] + 1e-16) + b2_ref[0]
    m = jnp.max(z, axis=1, keepdims=True)
    zz = z - m
    lse = jnp.log(jnp.sum(jnp.exp(zz), axis=1, keepdims=True))
    o_ref[...] = zz - lse


def _final(accp2, b2):
    blk = 1000
    return pl.pallas_call(
        _final_body,
        grid=(N // blk,),
        in_specs=[
            pl.BlockSpec((2, blk, 32), lambda i: (0, i, 0)),
            pl.BlockSpec((1, 16), lambda i: (0, 0)),
        ],
        out_specs=pl.BlockSpec((blk, 16), lambda i: (i, 0)),
        out_shape=jax.ShapeDtypeStruct((N, 16), jnp.float32),
    )(accp2, b2.reshape(1, 16))


def _edge_phase(table, adst, eidx3, ae_pk, width, logit_off, head_pairs,
                Bp, cps, nch, rem_tiles, split_out=False):
    mesh = plsc.VectorSubcoreMesh(core_axis_name="c", subcore_axis_name="s")

    if split_out:
        out_type = [jax.ShapeDtypeStruct((NC, N, logit_off), jnp.float32),
                    jax.ShapeDtypeStruct((NC, N, width - logit_off), jnp.float32)]
    else:
        out_type = jax.ShapeDtypeStruct((NC, N, width), jnp.float32)

    @functools.partial(
        pl.kernel,
        out_type=out_type,
        mesh=mesh,
        compiler_params=pltpu.CompilerParams(use_tc_tiling_on_sc=False),
    scratch_types=[
            pltpu.VMEM((2, cps, Bp), jnp.int32),
            pltpu.VMEM((2, cps, Bp), jnp.int32),
            pltpu.VMEM((cps * Bp // 8, 128), jnp.float32),
            pltpu.VMEM((Bp, width), jnp.float32),
            pltpu.VMEM((Bp, width), jnp.float32),
            pltpu.VMEM((Bp, 16), jnp.float32),
            pltpu.VMEM((Bp, 16), jnp.float32),
            pltpu.VMEM_SHARED((N, width), jnp.float32),
            pltpu.SemaphoreType.DMA,
            pltpu.SemaphoreType.DMA,
            pltpu.SemaphoreType.DMA,
            pltpu.SemaphoreType.DMA,
        ],
    )
    def k(tab, ad, eidx, aer, *rest):
        nout = 2 if split_out else 1
        outs = rest[:nout]
        (sciA, sciB, scae, rows0, rows1, d0, d1, shacc,
         semg0, semg1, sems0, sems1) = rest[nout:]
        c = lax.axis_index("c")
        s = lax.axis_index("s")

        zv = jnp.zeros((16,), jnp.float32)

        @plsc.parallel_loop(0, Bp, unroll=4)
        def _(e):
            for g in range(width // 16):
                rows0[e, pl.ds(g * 16, 16)] = zv

        nz = STRIPE // Bp
        rz = STRIPE - nz * Bp

        @pl.loop(0, nz)
        def _(i):
            pltpu.sync_copy(rows0, shacc.at[pl.ds(s * STRIPE + i * Bp, Bp)])

        if rz:
            pltpu.sync_copy(rows0.at[pl.ds(0, rz)],
                            shacc.at[pl.ds(s * STRIPE + nz * Bp, rz)])

        @pl.when(s == 0)
        def _():
            pltpu.sync_copy(rows0.at[pl.ds(0, TAIL)],
                            shacc.at[pl.ds(TAIL_OFF, TAIL)])

        plsc.subcore_barrier()
        tile = c * NS + s
        chunk0 = tile * nch + jnp.minimum(tile, rem_tiles)

        rpc = Bp // 8
        U = 2 * cps
        M = ((nch - cps) // U) * U
        TL = nch - M
        assert 0 < TL <= cps and (M // cps) % 2 == 0

        def copy_superchunk(sc, sci):
            g = chunk0 + sc * cps
            pltpu.sync_copy(eidx.at[:, pl.ds(g, cps)], sci)
            pltpu.sync_copy(aer.at[pl.ds(g * rpc, cps * rpc)], scae)

        def start_gather(m, sci, bufs):
            rows, d, gsem, _ = bufs
            pltpu.async_copy(tab.at[sci.at[0, m]], rows, gsem)
            pltpu.async_copy(ad.at[sci.at[1, m]], d, gsem)

        def wait_gather(bufs):
            rows, d, gsem, _ = bufs
            pltpu.make_async_copy(tab.at[sciA.at[0, 0]], rows, gsem).wait()
            pltpu.make_async_copy(ad.at[sciA.at[1, 0]], d, gsem).wait()

        def wait_scatter(bufs):
            rows, _, _, ssem = bufs
            pltpu.make_async_copy(rows, shacc.at[sciA.at[1, 0]], ssem).wait()

        def compute_scatter(m, sci, bufs, sync=False):
            rows, d, _, ssem = bufs

            @plsc.parallel_loop(0, Bp, unroll=4)
            def _(e):
                aev = scae[m * rpc + lax.div(e, 8), pl.ds(lax.rem(e, 8) * 16, 16)]
                logit = rows[e, pl.ds(logit_off, 16)] + d[e, :] + aev
                l = jnp.where(logit > 0, logit, logit * 0.2)
                ex = jnp.exp(l)
                rows[e, pl.ds(logit_off, 16)] = ex
                for grp, hl in head_pairs:
                    rows[e, pl.ds(grp * 16, 16)] = rows[e, pl.ds(grp * 16, 16)] * ex[hl]

            if sync:
                pltpu.sync_copy(rows, shacc.at[sci.at[1, m]], add=True)
            else:
                pltpu.async_copy(rows, shacc.at[sci.at[1, m]], ssem, add=True)

        bufA = (rows0, d0, semg0, sems0)
        bufB = (rows1, d1, semg1, sems1)
        copy_superchunk(0, sciA)
        start_gather(0, sciA, bufA)

        @pl.loop(0, M, step=U)
        def _(kk):
            for p in range(U):
                sci = sciA if p < cps else sciB
                bufs, obufs = (bufA, bufB) if p % 2 == 0 else (bufB, bufA)
                m = p % cps
                wait_gather(bufs)
                compute_scatter(m, sci, bufs)
                if p == 0:
                    @pl.when(kk > 0)
                    def _():
                        wait_scatter(obufs)
                else:
                    wait_scatter(obufs)
                if p == cps - 1:
                    copy_superchunk(lax.div(kk, cps) + 1, sciB)
                    start_gather(0, sciB, obufs)
                elif p == U - 1:
                    copy_superchunk(lax.div(kk, cps) + 2, sciA)
                    start_gather(0, sciA, obufs)
                else:
                    start_gather(m + 1, sci, obufs)

        for t in range(TL):
            bufs, obufs = (bufA, bufB) if t % 2 == 0 else (bufB, bufA)
            wait_gather(bufs)
            compute_scatter(t, sciA, bufs)
            if t < TL - 1:
                wait_scatter(obufs)
                start_gather(t + 1, sciA, obufs)

        wait_scatter(bufA)
        wait_scatter(bufB)

        if rem_tiles:
            @pl.when(tile < rem_tiles)
            def _():
                g = chunk0 + nch
                pltpu.sync_copy(eidx.at[:, pl.ds(g, 1)], sciA.at[:, pl.ds(0, 1)])
                pltpu.sync_copy(aer.at[pl.ds(g * rpc, rpc)], scae.at[pl.ds(0, rpc)])
                pltpu.sync_copy(tab.at[sciA.at[0, 0]], rows0)
                pltpu.sync_copy(ad.at[sciA.at[1, 0]], d0)
                compute_scatter(0, sciA, bufA, sync=True)

        plsc.subcore_barrier()

        def copy_out(lo, n):
            if split_out:
                pltpu.sync_copy(shacc.at[pl.ds(lo, n), pl.ds(0, logit_off)],
                                outs[0].at[c, pl.ds(lo, n)])
                pltpu.sync_copy(shacc.at[pl.ds(lo, n), pl.ds(logit_off, width - logit_off)],
                                outs[1].at[c, pl.ds(lo, n)])
            else:
                pltpu.sync_copy(shacc.at[pl.ds(lo, n)],
                                outs[0].at[c, pl.ds(lo, n)])

        copy_out(s * STRIPE, STRIPE)

        @pl.when(s == 0)
        def _():
            copy_out(TAIL_OFF, TAIL)

    return k(table, adst, eidx3, ae_pk)


def kernel(x, edge_index, edge_attr, W1, att_src1, att_dst1, We1, att_edge1,
           b1, W2, att_src2, att_dst2, We2, att_edge2, b2):
    f32 = jnp.float32
    Wsrc1 = _contract(W1, att_src1, H1, C1)
    Wdst1 = _contract(W1, att_dst1, H1, C1)
    Me1 = _contract(We1, att_edge1, H1, C1)
    Wsrc2 = _contract(W2, att_src2, 1, C2)
    Wdst2 = _contract(W2, att_dst2, 1, C2)
    Me2 = _contract(We2, att_edge2, 1, C2)
    z8 = jnp.zeros((D, 8), f32)
    z7 = jnp.zeros((D, 7), f32)
    wcat1 = jnp.concatenate([W1, Wsrc1, z8, Wdst1, z8], axis=1)
    m16 = jnp.concatenate([Me1, Me2, jnp.zeros((16, 7), f32)], axis=1)
    mblk = jnp.kron(jnp.eye(8, dtype=f32), m16)
    wcat2 = jnp.concatenate([W2, z8, Wsrc2, z7, z8, Wdst2, z7], axis=1)

    ei32 = edge_index.astype(jnp.int32)
    ea8 = edge_attr.reshape(E // 8, 128)

    table1, adst1, ae_pk = _prep_tables(x, wcat1, ea8, mblk)

    accm1, accd1 = _edge_phase(table1, adst1,
                               ei32.reshape(2, E // B1, B1), ae_pk,
                               width=144, logit_off=128,
                               head_pairs=[(h, h) for h in range(8)],
                               Bp=B1, cps=5, nch=(E // B1) // NW, rem_tiles=0,
                               split_out=True)

    table2, adst2 = _layer1_finish(accm1, accd1, b1, wcat2)

    nch2 = (E // B2) // NW
    accp2 = _edge_phase(table2, adst2,
                        ei32.reshape(2, E // B2, B2), ae_pk,
                        width=32, logit_off=16, head_pairs=[(0, 8)],
                        Bp=B2, cps=6, nch=nch2,
                        rem_tiles=(E // B2) - nch2 * NW)

    return _final(accp2, b2)
